# baseline (speedup 1.0000x reference)
import functools
import jax
import jax.numpy as jnp
import numpy as np
from jax import lax
from jax.experimental import pallas as pl
from jax.experimental.pallas import tpu as pltpu
from jax.experimental.pallas import tpu_sc as plsc

NUM_NODES = 10000
MAX_VISIT = 10
DECAY = 0.03
LAYERS = 3
HID = 128

NW = 32
NSUB = 16
CHUNK = 128


def _proj_body(tab_ref, w_ref, b_ref, o_ref):
    o_ref[...] = jnp.dot(tab_ref[...], w_ref[...],
                         preferred_element_type=jnp.float32) + b_ref[...]


def _project(tab, W, b):
    R = tab.shape[0]
    blk = 1280 if R % 1280 == 0 else R
    return pl.pallas_call(
        _proj_body,
        grid=(R // blk,),
        in_specs=[
            pl.BlockSpec((blk, 128), lambda i: (i, 0)),
            pl.BlockSpec((128, 128), lambda i: (0, 0)),
            pl.BlockSpec((1, 128), lambda i: (0, 0)),
        ],
        out_specs=pl.BlockSpec((blk, 128), lambda i: (i, 0)),
        out_shape=jax.ShapeDtypeStruct((R, 128), jnp.float32),
    )(tab, W, b.reshape(1, 128))


def _conv_body(agg_ref, x_ref, w_ref, b_ref, o_ref):
    acc = agg_ref[...] + x_ref[...]
    o_ref[...] = jnp.maximum(
        jnp.dot(acc, w_ref[...], preferred_element_type=jnp.float32)
        + b_ref[...], 0.0)


def _conv(agg, x, W, b):
    Tn = agg.shape[0]
    blk = 2000
    return pl.pallas_call(
        _conv_body,
        grid=(Tn // blk,),
        in_specs=[
            pl.BlockSpec((blk, 128), lambda i: (i, 0)),
            pl.BlockSpec((blk, 128), lambda i: (i, 0)),
            pl.BlockSpec((128, 128), lambda i: (0, 0)),
            pl.BlockSpec((1, 128), lambda i: (0, 0)),
        ],
        out_specs=pl.BlockSpec((blk, 128), lambda i: (i, 0)),
        out_shape=jax.ShapeDtypeStruct((Tn, 128), jnp.float32),
    )(agg, x, W, b.reshape(1, 128))


def _sc_mesh():
    return plsc.VectorSubcoreMesh(core_axis_name="c", subcore_axis_name="s")


def _sc_gather_rows(table, idx_pad, n_out):
    rows_total = idx_pad.shape[0]
    per_w = rows_total // NW
    nch = per_w // CHUNK

    def body(tab_hbm, idx_hbm, out_hbm, idxv, rowsv, sem):
        c = lax.axis_index("c")
        s = lax.axis_index("s")
        w = c * NSUB + s
        for j in range(nch):
            base = w * per_w + j * CHUNK
            pltpu.sync_copy(idx_hbm.at[pl.ds(base, CHUNK)], idxv)
            pltpu.async_copy(tab_hbm.at[idxv], rowsv, sem).wait()
            pltpu.sync_copy(rowsv, out_hbm.at[pl.ds(base, CHUNK)])

    out = pl.kernel(
        body,
        out_type=jax.ShapeDtypeStruct((rows_total, HID), jnp.float32),
        mesh=_sc_mesh(),
        scratch_types=[
            pltpu.VMEM((CHUNK,), jnp.int32),
            pltpu.VMEM((CHUNK, HID), jnp.float32),
            pltpu.SemaphoreType.DMA,
        ],
    )(table, idx_pad)
    return out[:n_out]


def _sc_message_pass(xtab, rtab, srcp, relp, dstlocp, attnl, wrell, ncha,
                    zblk, Tn):
    half = Tn // 2
    halfp = ((half + NSUB * 8 - 1) // (NSUB * 8)) * (NSUB * 8)
    stripe = halfp // NSUB
    Wcap = srcp.shape[0] // NW

    def body(xt_hbm, rt_hbm, src_hbm, rel_hbm, dst_hbm, att_hbm, wre_hbm,
             nch_hbm, z_hbm, agg_hbm,
             idxs, idxr, idxd, attv, wrev, rows, eatt, nchv, acc,
             sem1, sem2):
        c = lax.axis_index("c")
        s = lax.axis_index("s")
        w = c * NSUB + s
        pltpu.sync_copy(z_hbm, acc.at[pl.ds(s * stripe, stripe)])
        plsc.subcore_barrier()
        pltpu.sync_copy(nch_hbm.at[pl.ds(w * NSUB, NSUB)], nchv)
        n = nchv[...][0]

        def chunk_body(j, carry):
            base = w * Wcap + j * CHUNK
            pltpu.sync_copy(src_hbm.at[pl.ds(base, CHUNK)], idxs)
            pltpu.sync_copy(rel_hbm.at[pl.ds(base, CHUNK)], idxr)
            pltpu.sync_copy(dst_hbm.at[pl.ds(base, CHUNK)], idxd)
            pltpu.sync_copy(att_hbm.at[pl.ds(base, CHUNK)], attv)
            pltpu.sync_copy(wre_hbm.at[pl.ds(base, CHUNK)], wrev)
            cp1 = pltpu.async_copy(xt_hbm.at[idxs], rows, sem1)
            cp2 = pltpu.async_copy(rt_hbm.at[idxr], eatt, sem2)
            cp1.wait()
            cp2.wait()

            def group_body(g, carry2):
                av16 = attv[pl.ds(g * 16, 16)]
                wv16 = wrev[pl.ds(g * 16, 16)]
                for k in range(16):
                    e = g * 16 + k
                    av = jnp.full((16,), av16[k], jnp.float32)
                    wv = jnp.full((16,), wv16[k], jnp.float32)
                    for t in range(HID // 16):
                        sl = pl.ds(t * 16, 16)
                        m = rows[e, sl] * av + wv * eatt[e, sl]
                        rows[e, sl] = jnp.maximum(m, 0.0)
                return carry2

            lax.fori_loop(0, CHUNK // 16, group_body, 0)
            pltpu.sync_copy(rows, acc.at[idxd], add=True)
            return carry

        lax.fori_loop(0, n, chunk_body, 0)
        plsc.subcore_barrier()
        gbase = c * halfp + s * stripe
        pltpu.sync_copy(acc.at[pl.ds(s * stripe, stripe)],
                        agg_hbm.at[pl.ds(gbase, stripe)])

    return pl.kernel(
        body,
        out_type=jax.ShapeDtypeStruct((2 * halfp, HID), jnp.float32),
        mesh=_sc_mesh(),
        scratch_types=[
            pltpu.VMEM((CHUNK,), jnp.int32),
            pltpu.VMEM((CHUNK,), jnp.int32),
            pltpu.VMEM((CHUNK,), jnp.int32),
            pltpu.VMEM((CHUNK,), jnp.float32),
            pltpu.VMEM((CHUNK,), jnp.float32),
            pltpu.VMEM((CHUNK, HID), jnp.float32),
            pltpu.VMEM((CHUNK, HID), jnp.float32),
            pltpu.VMEM((NSUB,), jnp.int32),
            pltpu.VMEM_SHARED((halfp, HID), jnp.float32),
            pltpu.SemaphoreType.DMA,
            pltpu.SemaphoreType.DMA,
        ],
    )(xtab, rtab, srcp, relp, dstlocp, attnl, wrell, ncha, zblk)


def kernel(node_ids, rel_ids, edge_index, batch, visit_node, ehr_nodes,
           node_emb_w, rel_emb_w, lin_W, lin_b, alpha_W, alpha_b,
           beta_W, beta_b, conv_W, conv_b, WR_W, WR_b):
    N = NUM_NODES
    V = MAX_VISIT
    Bsz = visit_node.shape[0]
    Tn = node_ids.shape[0]
    E = rel_ids.shape[0]
    half = Tn // 2

    nrel = rel_emb_w.shape[0]
    pad_r = (-(N + nrel)) % 1280
    tab = jnp.concatenate(
        [node_emb_w, jnp.pad(rel_emb_w, ((0, pad_r), (0, 0)))], axis=0)
    proj = _project(tab, lin_W, lin_b)
    proj_node = proj[:N]
    rtab = proj[N:N + 64]

    tpad = (-Tn) % (NW * CHUNK)
    node_ids_pad = jnp.pad(node_ids, (0, tpad)).astype(jnp.int32)
    x0 = _sc_gather_rows(proj_node, node_ids_pad, Tn)

    counts = jnp.bincount(batch, length=Bsz)
    starts = jnp.cumsum(counts) - counts
    pos = jnp.arange(Tn) - jnp.take(starts, batch)
    P = x0 @ alpha_W[:, :, 0].T
    s = jnp.zeros((Bsz, N, LAYERS), jnp.float32).at[batch, pos].set(
        P, mode='drop')
    z = (visit_node[:, :, :, None] * s[:, None, :, :]
         + alpha_b[:, 0].reshape(1, 1, 1, LAYERS))
    m = jnp.max(z, axis=1, keepdims=True)
    ez = jnp.exp(z - m)
    alpha0 = ez[:, 0] / jnp.sum(ez, axis=1)
    lam0 = np.float32(np.exp(DECAY * V))
    beta0 = jnp.tanh(visit_node[:, 0, :] @ beta_W[:, :, 0].T
                     + beta_b[None, :, 0]) * lam0
    attn0 = alpha0 * beta0[:, None, :]

    src = edge_index[0].astype(jnp.int32)
    dst = edge_index[1].astype(jnp.int32)
    attn_e = attn0[jnp.take(batch, src), jnp.take(node_ids, src), :]
    wrel_tab = (rtab @ WR_W[:, :, 0].T + WR_b[None, :, 0])
    wrel_e = jnp.take(wrel_tab, rel_ids, axis=0)

    Wcap = ((E + NW * CHUNK - 1) // (NW * CHUNK)) * CHUNK
    ce = (dst >= half).astype(jnp.int32)
    r1 = jnp.cumsum(ce)
    n1 = r1[-1]
    n0 = E - n1
    local = jnp.where(ce == 1, r1 - 1, jnp.arange(E, dtype=jnp.int32) - r1)
    n_c = jnp.stack([n0, n1])
    q_c = jnp.maximum((n_c + NSUB - 1) // NSUB, 1)
    qe = jnp.take(q_c, ce)
    wl = local // qe
    posn = local - wl * qe
    wglob = ce * NSUB + wl
    slot = wglob * Wcap + posn
    inv = jnp.zeros((NW * Wcap,), jnp.int32).at[slot].set(
        jnp.arange(E, dtype=jnp.int32))
    widx = jnp.arange(NW)
    wc = widx // NSUB
    wlv = widx % NSUB
    counts_w = jnp.clip(jnp.take(n_c, wc) - wlv * jnp.take(q_c, wc),
                        0, jnp.take(q_c, wc))
    nch_w = (counts_w + CHUNK - 1) // CHUNK
    ncha = jnp.broadcast_to(nch_w[:, None], (NW, NSUB)).astype(
        jnp.int32).reshape(-1)
    maskb = (jnp.arange(Wcap)[None, :] < counts_w[:, None])
    mask = maskb.reshape(-1)
    fmask = mask.astype(jnp.float32)

    srcp = jnp.take(src, inv)
    nsrcp = jnp.take(node_ids, srcp).astype(jnp.int32)
    relp = jnp.take(rel_ids, inv).astype(jnp.int32)
    dstlocp = jnp.where(mask, jnp.take(dst, inv)
                        - jnp.take(ce, inv) * half, 0).astype(jnp.int32)
    attnp = jnp.take(attn_e, inv, axis=0) * fmask[:, None]
    wrelp = jnp.take(wrel_e, inv, axis=0) * fmask[:, None]
    attnp = attnp.T
    wrelp = wrelp.T

    halfp = ((half + NSUB * 8 - 1) // (NSUB * 8)) * (NSUB * 8)
    zblk = jnp.zeros((halfp // NSUB, HID), jnp.float32)

    x = x0
    for l in range(LAYERS):
        xt = proj_node if l == 0 else x
        idx_l = nsrcp if l == 0 else srcp
        agg_pad = _sc_message_pass(xt, rtab, idx_l, relp, dstlocp,
                                   attnp[l], wrelp[l], ncha, zblk, Tn)
        agg = jnp.concatenate(
            [agg_pad[:half], agg_pad[halfp:halfp + half]], axis=0)
        x = _conv(agg, x, conv_W[l], conv_b[l])
    return x
